# baseline (speedup 1.0000x reference)
import functools

import jax
import jax.numpy as jnp
from jax import lax
from jax.experimental import pallas as pl
from jax.experimental.pallas import tpu as pltpu
from jax.experimental.pallas import tpu_sc as plsc

_F32 = jnp.float32
_NW = 32
_LI = 128
_NSEG = 10240
_TRASH = 10000


def _sc_mesh():
    return plsc.VectorSubcoreMesh(core_axis_name="c", subcore_axis_name="s")


def _sc_gather(table, idx2):
    nrows, li = idx2.shape
    w = table.shape[1]
    nper = nrows // _NW

    @functools.partial(
        pl.kernel, mesh=_sc_mesh(),
        out_type=jax.ShapeDtypeStruct((nrows * li, w), _F32),
        scratch_types=[pltpu.VMEM((nper, li), jnp.int32),
                       pltpu.VMEM((li, w), _F32),
                       pltpu.SemaphoreType.DMA],
    )
    def gk(table_hbm, idx_hbm, out_hbm, idx_v, buf, sem):
        wid = lax.axis_index("s") * 2 + lax.axis_index("c")
        rlo = wid * nper
        pltpu.sync_copy(idx_hbm.at[pl.ds(rlo, nper)], idx_v)

        def body(j, carry):
            pltpu.async_copy(table_hbm.at[idx_v.at[j]], buf, sem).wait()
            pltpu.sync_copy(buf, out_hbm.at[pl.ds((rlo + j) * li, li)])
            return carry

        lax.fori_loop(0, nper, body, 0)

    return gk(table, idx2)


def _sc_scatter_add(m, idx2, w):
    nrows, li = idx2.shape
    nper = nrows // _NW
    rpt = _NSEG // 16

    @functools.partial(
        pl.kernel, mesh=_sc_mesh(),
        out_type=jax.ShapeDtypeStruct((2, _NSEG, w), _F32),
        scratch_types=[pltpu.VMEM((nper, li), jnp.int32),
                       pltpu.VMEM((li, w), _F32),
                       pltpu.VMEM((32, w), _F32),
                       pltpu.VMEM_SHARED((_NSEG, w), _F32)],
    )
    def sk(m_hbm, idx_hbm, out_hbm, idx_v, mbuf, zbuf, shared):
        c = lax.axis_index("c")
        sid = lax.axis_index("s")
        wid = sid * 2 + c

        def zb(i, carry):
            r = i // (w // 16)
            col = (i % (w // 16)) * 16
            zbuf[r, pl.ds(col, 16)] = jnp.zeros((16,), _F32)
            return carry

        lax.fori_loop(0, 32 * (w // 16), zb, 0)

        zlo = sid * rpt

        def zs(k, carry):
            pltpu.sync_copy(zbuf, shared.at[pl.ds(zlo + k * 32, 32)])
            return carry

        lax.fori_loop(0, rpt // 32, zs, 0)
        plsc.subcore_barrier()

        rlo = wid * nper
        pltpu.sync_copy(idx_hbm.at[pl.ds(rlo, nper)], idx_v)

        def body(j, carry):
            pltpu.sync_copy(m_hbm.at[pl.ds((rlo + j) * li, li)], mbuf)
            pltpu.sync_copy(mbuf, shared.at[idx_v.at[j]], add=True)
            return carry

        lax.fori_loop(0, nper, body, 0)
        plsc.subcore_barrier()

        def wr(k, carry):
            lo2 = zlo + k * li
            pltpu.sync_copy(shared.at[pl.ds(lo2, li)], mbuf)
            pltpu.sync_copy(mbuf, out_hbm.at[c, pl.ds(lo2, li)])
            return carry

        lax.fori_loop(0, rpt // li, wr, 0)

    return sk(m, idx2)


def _sc_pos_rel(pos_s, pos_d, idxs2, idxd2):
    shared = pos_s is pos_d
    ns = pos_s.shape[0]
    nd = pos_d.shape[0]
    nrows, li = idxs2.shape
    nper = nrows // _NW

    plane_scratch = [pltpu.VMEM((ns,), _F32)] * 3
    if not shared:
        plane_scratch += [pltpu.VMEM((nd,), _F32)] * 3
    ins = tuple(pos_s[:, i] for i in range(3))
    if not shared:
        ins += tuple(pos_d[:, i] for i in range(3))

    @functools.partial(
        pl.kernel, mesh=_sc_mesh(),
        compiler_params=pltpu.CompilerParams(needs_layout_passes=False),
        out_type=jax.ShapeDtypeStruct((nrows * li, 16), _F32),
        scratch_types=plane_scratch + [
            pltpu.VMEM((nper, li), jnp.int32),
            pltpu.VMEM((nper, li), jnp.int32),
            pltpu.VMEM((li, 16), _F32)],
    )
    def pk(*refs):
        nplanes = 3 if shared else 6
        plane_h = refs[:nplanes]
        is_h, id_h, out_hbm = refs[nplanes], refs[nplanes + 1], refs[nplanes + 2]
        plane_v = refs[nplanes + 3:2 * nplanes + 3]
        is_v, id_v, rbuf = refs[-3], refs[-2], refs[-1]
        if shared:
            src_v = dst_v = plane_v
        else:
            src_v, dst_v = plane_v[:3], plane_v[3:]

        wid = lax.axis_index("s") * 2 + lax.axis_index("c")
        rlo = wid * nper
        for h, v in zip(plane_h, plane_v):
            pltpu.sync_copy(h, v)
        pltpu.sync_copy(is_h.at[pl.ds(rlo, nper)], is_v)
        pltpu.sync_copy(id_h.at[pl.ds(rlo, nper)], id_v)

        def zr(i, carry):
            rbuf[i, pl.ds(0, 16)] = jnp.zeros((16,), _F32)
            return carry

        lax.fori_loop(0, li, zr, 0)
        iota = lax.iota(jnp.int32, 16)

        def body(j, carry):
            for g in range(li // 16):
                si = is_v[j, pl.ds(g * 16, 16)]
                di = id_v[j, pl.ds(g * 16, 16)]
                ridx = iota + (g * 16)
                for comp in range(3):
                    vs = plsc.load_gather(src_v[comp], [si])
                    vd = plsc.load_gather(dst_v[comp], [di])
                    plsc.store_scatter(
                        rbuf, [ridx, jnp.full((16,), comp, jnp.int32)],
                        vd - vs)
            pltpu.sync_copy(rbuf, out_hbm.at[pl.ds((rlo + j) * li, li)])
            return carry

        lax.fori_loop(0, nper, body, 0)

    return pk(*ins, idxs2, idxd2)


def _node3_body(h_ref, add_ref, Win_ref, W1_ref, W2_ref, W3_ref,
                o1_ref, o2_ref, o3_ref, oh_ref):
    h = jnp.tanh(jnp.dot(h_ref[...], Win_ref[...],
                         preferred_element_type=_F32)) + add_ref[...]
    o1_ref[...] = jnp.dot(h, W1_ref[...], preferred_element_type=_F32)
    o2_ref[...] = jnp.dot(h, W2_ref[...], preferred_element_type=_F32)
    o3_ref[...] = jnp.dot(h, W3_ref[...], preferred_element_type=_F32)
    oh_ref[...] = h


def _node3(h, add, Win, W1, W2, W3, bn=2000):
    n, dh = h.shape
    row = pl.BlockSpec((bn, dh), lambda i: (i, 0))
    wsp = pl.BlockSpec((dh, dh), lambda i: (0, 0))
    out = jax.ShapeDtypeStruct((n, dh), _F32)
    return pl.pallas_call(
        _node3_body,
        grid=(n // bn,),
        in_specs=[row, row, wsp, wsp, wsp, wsp],
        out_specs=[row, row, row, row],
        out_shape=[out, out, out, out],
    )(h, add, Win, W1, W2, W3)


def _edge_common(za, zb, rel16, attr, wd, Wc):
    d2 = jnp.sum(rel16 * rel16, axis=1)
    z = (za + zb
         + d2[:, None] * wd[None, :]
         + jnp.dot(attr, Wc, preferred_element_type=_F32))
    return jax.nn.relu(z)


def _edge_coef_body(za_ref, zb_ref, rel_ref, attr_ref, wd_ref, Wc_ref,
                    wx_ref, out_ref):
    m = _edge_common(za_ref[...], zb_ref[...], rel_ref[...], attr_ref[...],
                     wd_ref[...], Wc_ref[...])
    coef = jnp.tanh(jnp.sum(m * wx_ref[...][None, :], axis=1))
    out_ref[...] = rel_ref[...] * coef[:, None]


def _edge_coef(za, zb, rel16, attr, wd, Wc, wx, be=4096):
    e = za.shape[0]
    de = attr.shape[1]
    return pl.pallas_call(
        _edge_coef_body,
        grid=(e // be,),
        in_specs=[pl.BlockSpec((be, 128), lambda i: (i, 0)),
                  pl.BlockSpec((be, 128), lambda i: (i, 0)),
                  pl.BlockSpec((be, 16), lambda i: (i, 0)),
                  pl.BlockSpec((be, de), lambda i: (i, 0)),
                  pl.BlockSpec((128,), lambda i: (0,)),
                  pl.BlockSpec((de, 128), lambda i: (0, 0)),
                  pl.BlockSpec((128,), lambda i: (0,))],
        out_specs=pl.BlockSpec((be, 16), lambda i: (i, 0)),
        out_shape=jax.ShapeDtypeStruct((e, 16), _F32),
    )(za, zb, rel16, attr, wd, Wc, wx)


def _edge_msg_body(za_ref, zb_ref, rel_ref, attr_ref, wd_ref, Wc_ref, m_ref):
    m_ref[...] = _edge_common(za_ref[...], zb_ref[...], rel_ref[...],
                              attr_ref[...], wd_ref[...], Wc_ref[...])


def _edge_msg(za, zb, rel16, attr, wd, Wc, be=4096):
    e = za.shape[0]
    de = attr.shape[1]
    return pl.pallas_call(
        _edge_msg_body,
        grid=(e // be,),
        in_specs=[pl.BlockSpec((be, 128), lambda i: (i, 0)),
                  pl.BlockSpec((be, 128), lambda i: (i, 0)),
                  pl.BlockSpec((be, 16), lambda i: (i, 0)),
                  pl.BlockSpec((be, de), lambda i: (i, 0)),
                  pl.BlockSpec((128,), lambda i: (0,)),
                  pl.BlockSpec((de, 128), lambda i: (0, 0))],
        out_specs=pl.BlockSpec((be, 128), lambda i: (i, 0)),
        out_shape=jax.ShapeDtypeStruct((e, 128), _F32),
    )(za, zb, rel16, attr, wd, Wc)


def _upd_proj_body(h_ref, s_ref, W1_ref, W2_ref, o_ref):
    hp = h_ref[...] + jnp.dot(s_ref[...], W1_ref[...],
                              preferred_element_type=_F32)
    o_ref[...] = jnp.dot(hp, W2_ref[...], preferred_element_type=_F32)


def _upd_proj(h, sm, W1, W2, bn=2000):
    n, dh = h.shape
    row = pl.BlockSpec((bn, dh), lambda i: (i, 0))
    wsp = pl.BlockSpec((dh, dh), lambda i: (0, 0))
    return pl.pallas_call(
        _upd_proj_body,
        grid=(n // bn,),
        in_specs=[row, row, wsp, wsp],
        out_specs=row,
        out_shape=jax.ShapeDtypeStruct((n, dh), _F32),
    )(h, sm, W1, W2)


def _pad_rows(e_pad, idx, fill):
    pad = jnp.full((e_pad - idx.shape[0],), fill, jnp.int32)
    return jnp.concatenate([idx, pad]).reshape(e_pad // _LI, _LI)


def _pad_attr(e_pad, attr):
    return jnp.concatenate(
        [attr, jnp.zeros((e_pad - attr.shape[0], attr.shape[1]), _F32)])


def kernel(lig_x, lig_h, poc_x, poc_h, lig_edge_index, lig_edge_attr,
           poc_edge_index, poc_edge_attr, cross_edge_index, cross_edge_attr,
           lig_batch, poc_batch, W_in, w_t, W_in_p, W_m1, w_x_l, W_p1, W_p2,
           W_c1, w_x_c):
    n_lig = lig_x.shape[0]
    n_poc = poc_x.shape[0]
    dh = lig_h.shape[1]
    n_graphs = 200
    e_lig = lig_edge_index.shape[1]
    e_poc = poc_edge_index.shape[1]
    e_cross = cross_edge_index.shape[1]
    blk = _NW * 8 * _LI
    ep_lig = ((e_lig + blk - 1) // blk) * blk
    ep_poc = ((e_poc + blk - 1) // blk) * blk
    ep_cross = ((e_cross + blk - 1) // blk) * blk

    k1, k2 = jax.random.split(jax.random.key(42))
    t_per_graph = jax.random.uniform(k1, (n_graphs,), dtype=_F32)
    t_atom = t_per_graph[lig_batch]
    x0 = jax.random.normal(k2, lig_x.shape, dtype=_F32)

    poc_sum = jax.ops.segment_sum(poc_x, poc_batch, num_segments=n_graphs)
    poc_count = jnp.maximum(
        jax.ops.segment_sum(jnp.ones((n_poc, 1), dtype=_F32), poc_batch,
                            num_segments=n_graphs), 1.0)
    poc_center = poc_sum / poc_count
    poc_x_c = poc_x - poc_center[poc_batch]
    lig_x1_c = lig_x - poc_center[lig_batch]
    t_col = t_atom[:, None]
    x_t = (1.0 - t_col) * x0 + t_col * lig_x1_c
    target = lig_x1_c - x0

    Wa_m, Wb_m, wd_m, Wc_m = W_m1[:dh], W_m1[dh:2*dh], W_m1[2*dh], W_m1[2*dh+1:]
    Wa_p, Wb_p, wd_p, Wc_p = W_p1[:dh], W_p1[dh:2*dh], W_p1[2*dh], W_p1[2*dh+1:]
    Wa_c, Wb_c, wd_c, Wc_c = W_c1[:dh], W_c1[dh:2*dh], W_c1[2*dh], W_c1[2*dh+1:]

    T = t_col * w_t[None, :]
    A_l, B_l, B_lc, _ = _node3(lig_h, T, W_in, Wa_m, Wb_m, Wb_c)
    Zp = jnp.zeros((n_poc, dh), dtype=_F32)
    A_p, B_p, _, h_p = _node3(poc_h, Zp, W_in_p, Wa_p, Wb_p, Wa_c)

    ps, pd = poc_edge_index[0], poc_edge_index[1]
    s, d = lig_edge_index[0], lig_edge_index[1]
    cs, cd = cross_edge_index[0], cross_edge_index[1]

    s2 = _pad_rows(ep_lig, s, 0)
    d2i = _pad_rows(ep_lig, d, 0)
    dsc = _pad_rows(ep_lig, d, _TRASH)
    ps2 = _pad_rows(ep_poc, ps, 0)
    pd2 = _pad_rows(ep_poc, pd, 0)
    pdsc = _pad_rows(ep_poc, jnp.minimum(pd, _TRASH), _TRASH)
    cs2 = _pad_rows(ep_cross, cs, 0)
    cd2 = _pad_rows(ep_cross, cd, 0)
    cdsc = _pad_rows(ep_cross, cd, _TRASH)

    za_p = _sc_gather(A_p, ps2)
    zb_p = _sc_gather(B_p, pd2)
    rel_p = _sc_pos_rel(poc_x_c, poc_x_c, ps2, pd2)
    m_p = _edge_msg(za_p, zb_p, rel_p, _pad_attr(ep_poc, poc_edge_attr),
                    wd_p, Wc_p)
    S2 = _sc_scatter_add(m_p, pdsc, 128)
    seg = S2[0, :n_lig] + S2[1, :n_lig]
    A_c = _upd_proj(h_p[:n_lig], seg, W_p2, Wa_c)

    za_l = _sc_gather(A_l, s2)
    zb_l = _sc_gather(B_l, d2i)
    rel_l = _sc_pos_rel(x_t, x_t, s2, d2i)
    ct_l = _edge_coef(za_l, zb_l, rel_l, _pad_attr(ep_lig, lig_edge_attr),
                      wd_m, Wc_m, w_x_l[:, 0])
    v_l = jax.ops.segment_sum(ct_l[:e_lig, :3], d, num_segments=n_lig)

    za_c = _sc_gather(A_c, cs2)
    zb_c = _sc_gather(B_lc, cd2)
    rel_c = _sc_pos_rel(poc_x_c[:n_lig], x_t, cs2, cd2)
    ct_c = _edge_coef(za_c, zb_c, rel_c, _pad_attr(ep_cross, cross_edge_attr),
                      wd_c, Wc_c, w_x_c[:, 0])
    v_c = jax.ops.segment_sum(ct_c[:e_cross, :3], cd, num_segments=n_lig)

    v = v_l + v_c
    return jnp.mean((v - target) ** 2)


# speedup vs baseline: 1.3968x; 1.3968x over previous
import functools

import jax
import jax.numpy as jnp
from jax import lax
from jax.experimental import pallas as pl
from jax.experimental.pallas import tpu as pltpu
from jax.experimental.pallas import tpu_sc as plsc

_F32 = jnp.float32
_NW = 32
_LI = 128
_NSEG = 10240
_TRASH = 10000


def _sc_mesh():
    return plsc.VectorSubcoreMesh(core_axis_name="c", subcore_axis_name="s")


def _sc_gather2(ta, tb, idxa2, idxb2):
    nrows, li = idxa2.shape
    w = ta.shape[1]
    nper = nrows // _NW
    out = jax.ShapeDtypeStruct((nrows * li, w), _F32)

    @functools.partial(
        pl.kernel, mesh=_sc_mesh(),
        out_type=[out, out],
        scratch_types=[pltpu.VMEM((nper, li), jnp.int32),
                       pltpu.VMEM((nper, li), jnp.int32),
                       pltpu.VMEM((li, w), _F32),
                       pltpu.VMEM((li, w), _F32),
                       pltpu.SemaphoreType.DMA, pltpu.SemaphoreType.DMA],
    )
    def gk(ta_h, tb_h, ia_h, ib_h, oa_h, ob_h, ia_v, ib_v, bufa, bufb,
           sa, sb):
        wid = lax.axis_index("s") * 2 + lax.axis_index("c")
        rlo = wid * nper
        pltpu.sync_copy(ia_h.at[pl.ds(rlo, nper)], ia_v)
        pltpu.sync_copy(ib_h.at[pl.ds(rlo, nper)], ib_v)

        def body(j, carry):
            ca = pltpu.async_copy(ta_h.at[ia_v.at[j]], bufa, sa)
            cb = pltpu.async_copy(tb_h.at[ib_v.at[j]], bufb, sb)
            ca.wait()
            cb.wait()
            rw = (rlo + j) * li
            pltpu.sync_copy(bufa, oa_h.at[pl.ds(rw, li)])
            pltpu.sync_copy(bufb, ob_h.at[pl.ds(rw, li)])
            return carry

        lax.fori_loop(0, nper, body, 0)

    return gk(ta, tb, idxa2, idxb2)


def _sc_scatter_add(m, idx2, w):
    nrows, li = idx2.shape
    nper = nrows // _NW
    rpt = _NSEG // 16

    @functools.partial(
        pl.kernel, mesh=_sc_mesh(),
        out_type=jax.ShapeDtypeStruct((2, _NSEG, w), _F32),
        scratch_types=[pltpu.VMEM((nper, li), jnp.int32),
                       pltpu.VMEM((li, w), _F32),
                       pltpu.VMEM((32, w), _F32),
                       pltpu.VMEM_SHARED((_NSEG, w), _F32)],
    )
    def sk(m_hbm, idx_hbm, out_hbm, idx_v, mbuf, zbuf, shared):
        c = lax.axis_index("c")
        sid = lax.axis_index("s")
        wid = sid * 2 + c

        def zb(i, carry):
            r = i // (w // 16)
            col = (i % (w // 16)) * 16
            zbuf[r, pl.ds(col, 16)] = jnp.zeros((16,), _F32)
            return carry

        lax.fori_loop(0, 32 * (w // 16), zb, 0)

        zlo = sid * rpt

        def zs(k, carry):
            pltpu.sync_copy(zbuf, shared.at[pl.ds(zlo + k * 32, 32)])
            return carry

        lax.fori_loop(0, rpt // 32, zs, 0)
        plsc.subcore_barrier()

        rlo = wid * nper
        pltpu.sync_copy(idx_hbm.at[pl.ds(rlo, nper)], idx_v)

        def body(j, carry):
            pltpu.sync_copy(m_hbm.at[pl.ds((rlo + j) * li, li)], mbuf)
            pltpu.sync_copy(mbuf, shared.at[idx_v.at[j]], add=True)
            return carry

        lax.fori_loop(0, nper, body, 0)
        plsc.subcore_barrier()

        def wr(k, carry):
            lo2 = zlo + k * li
            pltpu.sync_copy(shared.at[pl.ds(lo2, li)], mbuf)
            pltpu.sync_copy(mbuf, out_hbm.at[c, pl.ds(lo2, li)])
            return carry

        lax.fori_loop(0, rpt // li, wr, 0)

    return sk(m, idx2)


def _sc_pos_rel(pos_s, pos_d, idxs2, idxd2):
    shared = pos_s is pos_d
    ns = pos_s.shape[0]
    nd = pos_d.shape[0]
    nrows, li = idxs2.shape
    nper = nrows // _NW

    plane_scratch = [pltpu.VMEM((ns,), _F32)] * 3
    if not shared:
        plane_scratch += [pltpu.VMEM((nd,), _F32)] * 3
    ins = tuple(pos_s[:, i] for i in range(3))
    if not shared:
        ins += tuple(pos_d[:, i] for i in range(3))

    @functools.partial(
        pl.kernel, mesh=_sc_mesh(),
        compiler_params=pltpu.CompilerParams(needs_layout_passes=False),
        out_type=jax.ShapeDtypeStruct((nrows * li, 16), _F32),
        scratch_types=plane_scratch + [
            pltpu.VMEM((nper, li), jnp.int32),
            pltpu.VMEM((nper, li), jnp.int32),
            pltpu.VMEM((li, 16), _F32)],
    )
    def pk(*refs):
        nplanes = 3 if shared else 6
        plane_h = refs[:nplanes]
        is_h, id_h, out_hbm = refs[nplanes], refs[nplanes + 1], refs[nplanes + 2]
        plane_v = refs[nplanes + 3:2 * nplanes + 3]
        is_v, id_v, rbuf = refs[-3], refs[-2], refs[-1]
        if shared:
            src_v = dst_v = plane_v
        else:
            src_v, dst_v = plane_v[:3], plane_v[3:]

        wid = lax.axis_index("s") * 2 + lax.axis_index("c")
        rlo = wid * nper
        for h, v in zip(plane_h, plane_v):
            pltpu.sync_copy(h, v)
        pltpu.sync_copy(is_h.at[pl.ds(rlo, nper)], is_v)
        pltpu.sync_copy(id_h.at[pl.ds(rlo, nper)], id_v)

        def zr(i, carry):
            rbuf[i, pl.ds(0, 16)] = jnp.zeros((16,), _F32)
            return carry

        lax.fori_loop(0, li, zr, 0)
        iota = lax.iota(jnp.int32, 16)

        def body(j, carry):
            for g in range(li // 16):
                si = is_v[j, pl.ds(g * 16, 16)]
                di = id_v[j, pl.ds(g * 16, 16)]
                ridx = iota + (g * 16)
                for comp in range(3):
                    vs = plsc.load_gather(src_v[comp], [si])
                    vd = plsc.load_gather(dst_v[comp], [di])
                    plsc.store_scatter(
                        rbuf, [ridx, jnp.full((16,), comp, jnp.int32)],
                        vd - vs)
            pltpu.sync_copy(rbuf, out_hbm.at[pl.ds((rlo + j) * li, li)])
            return carry

        lax.fori_loop(0, nper, body, 0)

    return pk(*ins, idxs2, idxd2)


def _node3_body(h_ref, add_ref, Win_ref, W1_ref, W2_ref, W3_ref,
                o1_ref, o2_ref, o3_ref, oh_ref):
    h = jnp.tanh(jnp.dot(h_ref[...], Win_ref[...],
                         preferred_element_type=_F32)) + add_ref[...]
    o1_ref[...] = jnp.dot(h, W1_ref[...], preferred_element_type=_F32)
    o2_ref[...] = jnp.dot(h, W2_ref[...], preferred_element_type=_F32)
    o3_ref[...] = jnp.dot(h, W3_ref[...], preferred_element_type=_F32)
    oh_ref[...] = h


def _node3(h, add, Win, W1, W2, W3, bn=2000):
    n, dh = h.shape
    row = pl.BlockSpec((bn, dh), lambda i: (i, 0))
    wsp = pl.BlockSpec((dh, dh), lambda i: (0, 0))
    out = jax.ShapeDtypeStruct((n, dh), _F32)
    return pl.pallas_call(
        _node3_body,
        grid=(n // bn,),
        in_specs=[row, row, wsp, wsp, wsp, wsp],
        out_specs=[row, row, row, row],
        out_shape=[out, out, out, out],
    )(h, add, Win, W1, W2, W3)


def _edge_common(za, zb, rel16, attr, wd, Wc):
    d2 = jnp.sum(rel16 * rel16, axis=1)
    z = (za + zb
         + d2[:, None] * wd[None, :]
         + jnp.dot(attr, Wc, preferred_element_type=_F32))
    return jax.nn.relu(z)


def _edge_coef_body(za_ref, zb_ref, rel_ref, attr_ref, wd_ref, Wc_ref,
                    wx_ref, out_ref):
    m = _edge_common(za_ref[...], zb_ref[...], rel_ref[...], attr_ref[...],
                     wd_ref[...], Wc_ref[...])
    coef = jnp.tanh(jnp.sum(m * wx_ref[...][None, :], axis=1))
    out_ref[...] = rel_ref[...] * coef[:, None]


def _edge_coef(za, zb, rel16, attr, wd, Wc, wx, be=4096):
    e = za.shape[0]
    de = attr.shape[1]
    return pl.pallas_call(
        _edge_coef_body,
        grid=(e // be,),
        in_specs=[pl.BlockSpec((be, 128), lambda i: (i, 0)),
                  pl.BlockSpec((be, 128), lambda i: (i, 0)),
                  pl.BlockSpec((be, 16), lambda i: (i, 0)),
                  pl.BlockSpec((be, de), lambda i: (i, 0)),
                  pl.BlockSpec((128,), lambda i: (0,)),
                  pl.BlockSpec((de, 128), lambda i: (0, 0)),
                  pl.BlockSpec((128,), lambda i: (0,))],
        out_specs=pl.BlockSpec((be, 16), lambda i: (i, 0)),
        out_shape=jax.ShapeDtypeStruct((e, 16), _F32),
    )(za, zb, rel16, attr, wd, Wc, wx)


def _edge_msg_body(za_ref, zb_ref, rel_ref, attr_ref, wd_ref, Wc_ref, m_ref):
    m_ref[...] = _edge_common(za_ref[...], zb_ref[...], rel_ref[...],
                              attr_ref[...], wd_ref[...], Wc_ref[...])


def _edge_msg(za, zb, rel16, attr, wd, Wc, be=4096):
    e = za.shape[0]
    de = attr.shape[1]
    return pl.pallas_call(
        _edge_msg_body,
        grid=(e // be,),
        in_specs=[pl.BlockSpec((be, 128), lambda i: (i, 0)),
                  pl.BlockSpec((be, 128), lambda i: (i, 0)),
                  pl.BlockSpec((be, 16), lambda i: (i, 0)),
                  pl.BlockSpec((be, de), lambda i: (i, 0)),
                  pl.BlockSpec((128,), lambda i: (0,)),
                  pl.BlockSpec((de, 128), lambda i: (0, 0))],
        out_specs=pl.BlockSpec((be, 128), lambda i: (i, 0)),
        out_shape=jax.ShapeDtypeStruct((e, 128), _F32),
    )(za, zb, rel16, attr, wd, Wc)


def _upd_proj_body(h_ref, s_ref, W1_ref, W2_ref, o_ref):
    hp = h_ref[...] + jnp.dot(s_ref[...], W1_ref[...],
                              preferred_element_type=_F32)
    o_ref[...] = jnp.dot(hp, W2_ref[...], preferred_element_type=_F32)


def _upd_proj(h, sm, W1, W2, bn=2000):
    n, dh = h.shape
    row = pl.BlockSpec((bn, dh), lambda i: (i, 0))
    wsp = pl.BlockSpec((dh, dh), lambda i: (0, 0))
    return pl.pallas_call(
        _upd_proj_body,
        grid=(n // bn,),
        in_specs=[row, row, wsp, wsp],
        out_specs=row,
        out_shape=jax.ShapeDtypeStruct((n, dh), _F32),
    )(h, sm, W1, W2)


def _pad_rows(e_pad, idx, fill):
    pad = jnp.full((e_pad - idx.shape[0],), fill, jnp.int32)
    return jnp.concatenate([idx, pad]).reshape(e_pad // _LI, _LI)


def _pad_attr(e_pad, attr):
    return jnp.concatenate(
        [attr, jnp.zeros((e_pad - attr.shape[0], attr.shape[1]), _F32)])


def kernel(lig_x, lig_h, poc_x, poc_h, lig_edge_index, lig_edge_attr,
           poc_edge_index, poc_edge_attr, cross_edge_index, cross_edge_attr,
           lig_batch, poc_batch, W_in, w_t, W_in_p, W_m1, w_x_l, W_p1, W_p2,
           W_c1, w_x_c):
    n_lig = lig_x.shape[0]
    n_poc = poc_x.shape[0]
    dh = lig_h.shape[1]
    n_graphs = 200
    e_lig = lig_edge_index.shape[1]
    e_poc = poc_edge_index.shape[1]
    e_cross = cross_edge_index.shape[1]
    blk = _NW * 8 * _LI
    ep_lig = ((e_lig + blk - 1) // blk) * blk
    ep_poc = ((e_poc + blk - 1) // blk) * blk
    ep_cross = ((e_cross + blk - 1) // blk) * blk

    k1, k2 = jax.random.split(jax.random.key(42))
    t_per_graph = jax.random.uniform(k1, (n_graphs,), dtype=_F32)
    t_atom = t_per_graph[lig_batch]
    x0 = jax.random.normal(k2, lig_x.shape, dtype=_F32)

    poc_sum = jax.ops.segment_sum(poc_x, poc_batch, num_segments=n_graphs)
    poc_count = jnp.maximum(
        jax.ops.segment_sum(jnp.ones((n_poc, 1), dtype=_F32), poc_batch,
                            num_segments=n_graphs), 1.0)
    poc_center = poc_sum / poc_count
    poc_x_c = poc_x - poc_center[poc_batch]
    lig_x1_c = lig_x - poc_center[lig_batch]
    t_col = t_atom[:, None]
    x_t = (1.0 - t_col) * x0 + t_col * lig_x1_c
    target = lig_x1_c - x0

    Wa_m, Wb_m, wd_m, Wc_m = W_m1[:dh], W_m1[dh:2*dh], W_m1[2*dh], W_m1[2*dh+1:]
    Wa_p, Wb_p, wd_p, Wc_p = W_p1[:dh], W_p1[dh:2*dh], W_p1[2*dh], W_p1[2*dh+1:]
    Wa_c, Wb_c, wd_c, Wc_c = W_c1[:dh], W_c1[dh:2*dh], W_c1[2*dh], W_c1[2*dh+1:]

    T = t_col * w_t[None, :]
    A_l, B_l, B_lc, _ = _node3(lig_h, T, W_in, Wa_m, Wb_m, Wb_c)
    Zp = jnp.zeros((n_poc, dh), dtype=_F32)
    A_p, B_p, _, h_p = _node3(poc_h, Zp, W_in_p, Wa_p, Wb_p, Wa_c)

    ps, pd = poc_edge_index[0], poc_edge_index[1]
    s, d = lig_edge_index[0], lig_edge_index[1]
    cs, cd = cross_edge_index[0], cross_edge_index[1]

    s2 = _pad_rows(ep_lig, s, 0)
    d2i = _pad_rows(ep_lig, d, 0)
    dsc = _pad_rows(ep_lig, d, _TRASH)
    ps2 = _pad_rows(ep_poc, ps, 0)
    pd2 = _pad_rows(ep_poc, pd, 0)
    pdsc = _pad_rows(ep_poc, jnp.minimum(pd, _TRASH), _TRASH)
    cs2 = _pad_rows(ep_cross, cs, 0)
    cd2 = _pad_rows(ep_cross, cd, 0)
    cdsc = _pad_rows(ep_cross, cd, _TRASH)

    za_p, zb_p = _sc_gather2(A_p, B_p, ps2, pd2)
    rel_p = _sc_pos_rel(poc_x_c, poc_x_c, ps2, pd2)
    m_p = _edge_msg(za_p, zb_p, rel_p, _pad_attr(ep_poc, poc_edge_attr),
                    wd_p, Wc_p)
    S2 = _sc_scatter_add(m_p, pdsc, 128)
    seg = S2[0, :n_lig] + S2[1, :n_lig]
    A_c = _upd_proj(h_p[:n_lig], seg, W_p2, Wa_c)

    za_l, zb_l = _sc_gather2(A_l, B_l, s2, d2i)
    rel_l = _sc_pos_rel(x_t, x_t, s2, d2i)
    ct_l = _edge_coef(za_l, zb_l, rel_l, _pad_attr(ep_lig, lig_edge_attr),
                      wd_m, Wc_m, w_x_l[:, 0])
    v_l = jax.ops.segment_sum(ct_l[:e_lig, :3], d, num_segments=n_lig)

    za_c, zb_c = _sc_gather2(A_c, B_lc, cs2, cd2)
    rel_c = _sc_pos_rel(poc_x_c[:n_lig], x_t, cs2, cd2)
    ct_c = _edge_coef(za_c, zb_c, rel_c, _pad_attr(ep_cross, cross_edge_attr),
                      wd_c, Wc_c, w_x_c[:, 0])
    v_c = jax.ops.segment_sum(ct_c[:e_cross, :3], cd, num_segments=n_lig)

    v = v_l + v_c
    return jnp.mean((v - target) ** 2)


# speedup vs baseline: 1.4526x; 1.0399x over previous
import functools

import jax
import jax.numpy as jnp
from jax import lax
from jax.experimental import pallas as pl
from jax.experimental.pallas import tpu as pltpu
from jax.experimental.pallas import tpu_sc as plsc

_F32 = jnp.float32
_NW = 32
_LI = 128
_NSEG = 10240
_TRASH = 10000


def _sc_mesh():
    return plsc.VectorSubcoreMesh(core_axis_name="c", subcore_axis_name="s")


def _sc_gather2(ta, tb, idxa2, idxb2):
    nrows, li = idxa2.shape
    w = ta.shape[1]
    nper = nrows // _NW
    out = jax.ShapeDtypeStruct((nrows * li, w), _F32)

    @functools.partial(
        pl.kernel, mesh=_sc_mesh(),
        out_type=[out, out],
        scratch_types=[pltpu.VMEM((nper, li), jnp.int32),
                       pltpu.VMEM((nper, li), jnp.int32),
                       pltpu.VMEM((2, li, w), _F32),
                       pltpu.VMEM((2, li, w), _F32)]
                      + [pltpu.SemaphoreType.DMA] * 8,
    )
    def gk(ta_h, tb_h, ia_h, ib_h, oa_h, ob_h, ia_v, ib_v, bufa, bufb,
           ga0, ga1, gb0, gb1, wa0, wa1, wb0, wb1):
        gsa, gsb, wsa, wsb = (ga0, ga1), (gb0, gb1), (wa0, wa1), (wb0, wb1)
        wid = lax.axis_index("s") * 2 + lax.axis_index("c")
        rlo = wid * nper
        pltpu.sync_copy(ia_h.at[pl.ds(rlo, nper)], ia_v)
        pltpu.sync_copy(ib_h.at[pl.ds(rlo, nper)], ib_v)
        pltpu.async_copy(ta_h.at[ia_v.at[0]], bufa.at[0], gsa[0])
        pltpu.async_copy(tb_h.at[ib_v.at[0]], bufb.at[0], gsb[0])

        def pair(j0, carry):
            for b in (0, 1):
                j = j0 * 2 + b
                ob = 1 - b
                pltpu.make_async_copy(ta_h.at[ia_v.at[j]], bufa.at[b],
                                      gsa[b]).wait()
                pltpu.make_async_copy(tb_h.at[ib_v.at[j]], bufb.at[b],
                                      gsb[b]).wait()

                @pl.when(j >= 1)
                def _():
                    rw = (rlo + j - 1) * li
                    pltpu.make_async_copy(
                        bufa.at[ob], oa_h.at[pl.ds(rw, li)], wsa[ob]).wait()
                    pltpu.make_async_copy(
                        bufb.at[ob], ob_h.at[pl.ds(rw, li)], wsb[ob]).wait()

                @pl.when(j + 1 < nper)
                def _():
                    pltpu.async_copy(ta_h.at[ia_v.at[j + 1]], bufa.at[ob],
                                     gsa[ob])
                    pltpu.async_copy(tb_h.at[ib_v.at[j + 1]], bufb.at[ob],
                                     gsb[ob])

                rw = (rlo + j) * li
                pltpu.async_copy(bufa.at[b], oa_h.at[pl.ds(rw, li)], wsa[b])
                pltpu.async_copy(bufb.at[b], ob_h.at[pl.ds(rw, li)], wsb[b])
            return carry

        lax.fori_loop(0, nper // 2, pair, 0)
        rw = (rlo + nper - 1) * li
        lb = (nper - 1) % 2
        pltpu.make_async_copy(bufa.at[lb], oa_h.at[pl.ds(rw, li)],
                              wsa[lb]).wait()
        pltpu.make_async_copy(bufb.at[lb], ob_h.at[pl.ds(rw, li)],
                              wsb[lb]).wait()

    return gk(ta, tb, idxa2, idxb2)


def _sc_scatter_add(m, idx2, w):
    nrows, li = idx2.shape
    nper = nrows // _NW
    rpt = _NSEG // 16

    @functools.partial(
        pl.kernel, mesh=_sc_mesh(),
        out_type=jax.ShapeDtypeStruct((2, _NSEG, w), _F32),
        scratch_types=[pltpu.VMEM((nper, li), jnp.int32),
                       pltpu.VMEM((2, li, w), _F32),
                       pltpu.VMEM((32, w), _F32),
                       pltpu.VMEM_SHARED((_NSEG, w), _F32)]
                      + [pltpu.SemaphoreType.DMA] * 2,
    )
    def sk(m_hbm, idx_hbm, out_hbm, idx_v, mbuf, zbuf, shared, ls0, ls1):
        lsem = (ls0, ls1)
        c = lax.axis_index("c")
        sid = lax.axis_index("s")
        wid = sid * 2 + c

        def zb(i, carry):
            r = i // (w // 16)
            col = (i % (w // 16)) * 16
            zbuf[r, pl.ds(col, 16)] = jnp.zeros((16,), _F32)
            return carry

        lax.fori_loop(0, 32 * (w // 16), zb, 0)

        zlo = sid * rpt

        def zs(k, carry):
            pltpu.sync_copy(zbuf, shared.at[pl.ds(zlo + k * 32, 32)])
            return carry

        lax.fori_loop(0, rpt // 32, zs, 0)

        rlo = wid * nper
        pltpu.sync_copy(idx_hbm.at[pl.ds(rlo, nper)], idx_v)
        plsc.subcore_barrier()
        pltpu.async_copy(m_hbm.at[pl.ds(rlo * li, li)], mbuf.at[0], lsem[0])

        def pair(j0, carry):
            for b in (0, 1):
                j = j0 * 2 + b
                pltpu.make_async_copy(m_hbm.at[pl.ds((rlo + j) * li, li)],
                                      mbuf.at[b], lsem[b]).wait()

                @pl.when(j + 1 < nper)
                def _():
                    pltpu.async_copy(
                        m_hbm.at[pl.ds((rlo + j + 1) * li, li)],
                        mbuf.at[1 - b], lsem[1 - b])

                pltpu.sync_copy(mbuf.at[b], shared.at[idx_v.at[j]], add=True)
            return carry

        lax.fori_loop(0, nper // 2, pair, 0)
        plsc.subcore_barrier()

        def wr(k, carry):
            lo2 = zlo + k * li
            pltpu.sync_copy(shared.at[pl.ds(lo2, li)], mbuf.at[0])
            pltpu.sync_copy(mbuf.at[0], out_hbm.at[c, pl.ds(lo2, li)])
            return carry

        lax.fori_loop(0, rpt // li, wr, 0)

    return sk(m, idx2)


def _sc_pos_rel(pos_s, pos_d, idxs2, idxd2):
    shared = pos_s is pos_d
    ns = pos_s.shape[0]
    nd = pos_d.shape[0]
    nrows, li = idxs2.shape
    nper = nrows // _NW

    plane_scratch = [pltpu.VMEM((ns,), _F32)] * 3
    if not shared:
        plane_scratch += [pltpu.VMEM((nd,), _F32)] * 3
    ins = tuple(pos_s[:, i] for i in range(3))
    if not shared:
        ins += tuple(pos_d[:, i] for i in range(3))

    @functools.partial(
        pl.kernel, mesh=_sc_mesh(),
        compiler_params=pltpu.CompilerParams(needs_layout_passes=False),
        out_type=jax.ShapeDtypeStruct((nrows * li, 16), _F32),
        scratch_types=plane_scratch + [
            pltpu.VMEM((nper, li), jnp.int32),
            pltpu.VMEM((nper, li), jnp.int32),
            pltpu.VMEM((2, li, 16), _F32),
            pltpu.SemaphoreType.DMA, pltpu.SemaphoreType.DMA],
    )
    def pk(*refs):
        nplanes = 3 if shared else 6
        plane_h = refs[:nplanes]
        is_h, id_h, out_hbm = refs[nplanes], refs[nplanes + 1], refs[nplanes + 2]
        plane_v = refs[nplanes + 3:2 * nplanes + 3]
        is_v, id_v, rbuf = refs[-5], refs[-4], refs[-3]
        wsem = (refs[-2], refs[-1])
        if shared:
            src_v = dst_v = plane_v
        else:
            src_v, dst_v = plane_v[:3], plane_v[3:]

        wid = lax.axis_index("s") * 2 + lax.axis_index("c")
        rlo = wid * nper
        for h, v in zip(plane_h, plane_v):
            pltpu.sync_copy(h, v)
        pltpu.sync_copy(is_h.at[pl.ds(rlo, nper)], is_v)
        pltpu.sync_copy(id_h.at[pl.ds(rlo, nper)], id_v)

        def zr(i, carry):
            rbuf[i // li, i % li, pl.ds(0, 16)] = jnp.zeros((16,), _F32)
            return carry

        lax.fori_loop(0, 2 * li, zr, 0)
        iota = lax.iota(jnp.int32, 16)

        def pair(j0, carry):
            for b in (0, 1):
                j = j0 * 2 + b

                @pl.when(j >= 2)
                def _():
                    pltpu.make_async_copy(
                        rbuf.at[b],
                        out_hbm.at[pl.ds((rlo + j - 2) * li, li)],
                        wsem[b]).wait()

                for g in range(li // 16):
                    si = is_v[j, pl.ds(g * 16, 16)]
                    di = id_v[j, pl.ds(g * 16, 16)]
                    ridx = iota + (g * 16)
                    for comp in range(3):
                        vs = plsc.load_gather(src_v[comp], [si])
                        vd = plsc.load_gather(dst_v[comp], [di])
                        plsc.store_scatter(
                            rbuf.at[b],
                            [ridx, jnp.full((16,), comp, jnp.int32)],
                            vd - vs)
                pltpu.async_copy(rbuf.at[b],
                                 out_hbm.at[pl.ds((rlo + j) * li, li)],
                                 wsem[b])
            return carry

        lax.fori_loop(0, nper // 2, pair, 0)
        for b in (0, 1):
            j = nper - 2 + b
            pltpu.make_async_copy(
                rbuf.at[b], out_hbm.at[pl.ds((rlo + j) * li, li)],
                wsem[b]).wait()

    return pk(*ins, idxs2, idxd2)


def _node3_body(h_ref, add_ref, Win_ref, W1_ref, W2_ref, W3_ref,
                o1_ref, o2_ref, o3_ref, oh_ref):
    h = jnp.tanh(jnp.dot(h_ref[...], Win_ref[...],
                         preferred_element_type=_F32)) + add_ref[...]
    o1_ref[...] = jnp.dot(h, W1_ref[...], preferred_element_type=_F32)
    o2_ref[...] = jnp.dot(h, W2_ref[...], preferred_element_type=_F32)
    o3_ref[...] = jnp.dot(h, W3_ref[...], preferred_element_type=_F32)
    oh_ref[...] = h


def _node3(h, add, Win, W1, W2, W3, bn=2000):
    n, dh = h.shape
    row = pl.BlockSpec((bn, dh), lambda i: (i, 0))
    wsp = pl.BlockSpec((dh, dh), lambda i: (0, 0))
    out = jax.ShapeDtypeStruct((n, dh), _F32)
    return pl.pallas_call(
        _node3_body,
        grid=(n // bn,),
        in_specs=[row, row, wsp, wsp, wsp, wsp],
        out_specs=[row, row, row, row],
        out_shape=[out, out, out, out],
    )(h, add, Win, W1, W2, W3)


def _edge_common(za, zb, rel16, attr, wd, Wc):
    d2 = jnp.sum(rel16 * rel16, axis=1)
    z = (za + zb
         + d2[:, None] * wd[None, :]
         + jnp.dot(attr, Wc, preferred_element_type=_F32))
    return jax.nn.relu(z)


def _edge_coef_body(za_ref, zb_ref, rel_ref, attr_ref, wd_ref, Wc_ref,
                    wx_ref, out_ref):
    m = _edge_common(za_ref[...], zb_ref[...], rel_ref[...], attr_ref[...],
                     wd_ref[...], Wc_ref[...])
    coef = jnp.tanh(jnp.sum(m * wx_ref[...][None, :], axis=1))
    out_ref[...] = rel_ref[...] * coef[:, None]


def _edge_coef(za, zb, rel16, attr, wd, Wc, wx, be=4096):
    e = za.shape[0]
    de = attr.shape[1]
    return pl.pallas_call(
        _edge_coef_body,
        grid=(e // be,),
        in_specs=[pl.BlockSpec((be, 128), lambda i: (i, 0)),
                  pl.BlockSpec((be, 128), lambda i: (i, 0)),
                  pl.BlockSpec((be, 16), lambda i: (i, 0)),
                  pl.BlockSpec((be, de), lambda i: (i, 0)),
                  pl.BlockSpec((128,), lambda i: (0,)),
                  pl.BlockSpec((de, 128), lambda i: (0, 0)),
                  pl.BlockSpec((128,), lambda i: (0,))],
        out_specs=pl.BlockSpec((be, 16), lambda i: (i, 0)),
        out_shape=jax.ShapeDtypeStruct((e, 16), _F32),
    )(za, zb, rel16, attr, wd, Wc, wx)


def _edge_msg_body(za_ref, zb_ref, rel_ref, attr_ref, wd_ref, Wc_ref, m_ref):
    m_ref[...] = _edge_common(za_ref[...], zb_ref[...], rel_ref[...],
                              attr_ref[...], wd_ref[...], Wc_ref[...])


def _edge_msg(za, zb, rel16, attr, wd, Wc, be=4096):
    e = za.shape[0]
    de = attr.shape[1]
    return pl.pallas_call(
        _edge_msg_body,
        grid=(e // be,),
        in_specs=[pl.BlockSpec((be, 128), lambda i: (i, 0)),
                  pl.BlockSpec((be, 128), lambda i: (i, 0)),
                  pl.BlockSpec((be, 16), lambda i: (i, 0)),
                  pl.BlockSpec((be, de), lambda i: (i, 0)),
                  pl.BlockSpec((128,), lambda i: (0,)),
                  pl.BlockSpec((de, 128), lambda i: (0, 0))],
        out_specs=pl.BlockSpec((be, 128), lambda i: (i, 0)),
        out_shape=jax.ShapeDtypeStruct((e, 128), _F32),
    )(za, zb, rel16, attr, wd, Wc)


def _upd_proj_body(h_ref, s_ref, W1_ref, W2_ref, o_ref):
    hp = h_ref[...] + jnp.dot(s_ref[...], W1_ref[...],
                              preferred_element_type=_F32)
    o_ref[...] = jnp.dot(hp, W2_ref[...], preferred_element_type=_F32)


def _upd_proj(h, sm, W1, W2, bn=2000):
    n, dh = h.shape
    row = pl.BlockSpec((bn, dh), lambda i: (i, 0))
    wsp = pl.BlockSpec((dh, dh), lambda i: (0, 0))
    return pl.pallas_call(
        _upd_proj_body,
        grid=(n // bn,),
        in_specs=[row, row, wsp, wsp],
        out_specs=row,
        out_shape=jax.ShapeDtypeStruct((n, dh), _F32),
    )(h, sm, W1, W2)


def _pad_rows(e_pad, idx, fill):
    pad = jnp.full((e_pad - idx.shape[0],), fill, jnp.int32)
    return jnp.concatenate([idx, pad]).reshape(e_pad // _LI, _LI)


def _pad_attr(e_pad, attr):
    return jnp.concatenate(
        [attr, jnp.zeros((e_pad - attr.shape[0], attr.shape[1]), _F32)])


def kernel(lig_x, lig_h, poc_x, poc_h, lig_edge_index, lig_edge_attr,
           poc_edge_index, poc_edge_attr, cross_edge_index, cross_edge_attr,
           lig_batch, poc_batch, W_in, w_t, W_in_p, W_m1, w_x_l, W_p1, W_p2,
           W_c1, w_x_c):
    n_lig = lig_x.shape[0]
    n_poc = poc_x.shape[0]
    dh = lig_h.shape[1]
    n_graphs = 200
    e_lig = lig_edge_index.shape[1]
    e_poc = poc_edge_index.shape[1]
    e_cross = cross_edge_index.shape[1]
    blk = _NW * 8 * _LI
    ep_lig = ((e_lig + blk - 1) // blk) * blk
    ep_poc = ((e_poc + blk - 1) // blk) * blk
    ep_cross = ((e_cross + blk - 1) // blk) * blk

    k1, k2 = jax.random.split(jax.random.key(42))
    t_per_graph = jax.random.uniform(k1, (n_graphs,), dtype=_F32)
    t_atom = t_per_graph[lig_batch]
    x0 = jax.random.normal(k2, lig_x.shape, dtype=_F32)

    poc_sum = jax.ops.segment_sum(poc_x, poc_batch, num_segments=n_graphs)
    poc_count = jnp.maximum(
        jax.ops.segment_sum(jnp.ones((n_poc, 1), dtype=_F32), poc_batch,
                            num_segments=n_graphs), 1.0)
    poc_center = poc_sum / poc_count
    poc_x_c = poc_x - poc_center[poc_batch]
    lig_x1_c = lig_x - poc_center[lig_batch]
    t_col = t_atom[:, None]
    x_t = (1.0 - t_col) * x0 + t_col * lig_x1_c
    target = lig_x1_c - x0

    Wa_m, Wb_m, wd_m, Wc_m = W_m1[:dh], W_m1[dh:2*dh], W_m1[2*dh], W_m1[2*dh+1:]
    Wa_p, Wb_p, wd_p, Wc_p = W_p1[:dh], W_p1[dh:2*dh], W_p1[2*dh], W_p1[2*dh+1:]
    Wa_c, Wb_c, wd_c, Wc_c = W_c1[:dh], W_c1[dh:2*dh], W_c1[2*dh], W_c1[2*dh+1:]

    T = t_col * w_t[None, :]
    A_l, B_l, B_lc, _ = _node3(lig_h, T, W_in, Wa_m, Wb_m, Wb_c)
    Zp = jnp.zeros((n_poc, dh), dtype=_F32)
    A_p, B_p, _, h_p = _node3(poc_h, Zp, W_in_p, Wa_p, Wb_p, Wa_c)

    ps, pd = poc_edge_index[0], poc_edge_index[1]
    s, d = lig_edge_index[0], lig_edge_index[1]
    cs, cd = cross_edge_index[0], cross_edge_index[1]

    s2 = _pad_rows(ep_lig, s, 0)
    d2i = _pad_rows(ep_lig, d, 0)
    dsc = _pad_rows(ep_lig, d, _TRASH)
    ps2 = _pad_rows(ep_poc, ps, 0)
    pd2 = _pad_rows(ep_poc, pd, 0)
    pdsc = _pad_rows(ep_poc, jnp.minimum(pd, _TRASH), _TRASH)
    cs2 = _pad_rows(ep_cross, cs, 0)
    cd2 = _pad_rows(ep_cross, cd, 0)
    cdsc = _pad_rows(ep_cross, cd, _TRASH)

    za_p, zb_p = _sc_gather2(A_p, B_p, ps2, pd2)
    rel_p = _sc_pos_rel(poc_x_c, poc_x_c, ps2, pd2)
    m_p = _edge_msg(za_p, zb_p, rel_p, _pad_attr(ep_poc, poc_edge_attr),
                    wd_p, Wc_p)
    S2 = _sc_scatter_add(m_p, pdsc, 128)
    seg = S2[0, :n_lig] + S2[1, :n_lig]
    A_c = _upd_proj(h_p[:n_lig], seg, W_p2, Wa_c)

    za_l, zb_l = _sc_gather2(A_l, B_l, s2, d2i)
    rel_l = _sc_pos_rel(x_t, x_t, s2, d2i)
    ct_l = _edge_coef(za_l, zb_l, rel_l, _pad_attr(ep_lig, lig_edge_attr),
                      wd_m, Wc_m, w_x_l[:, 0])
    v_l = jax.ops.segment_sum(ct_l[:e_lig, :3], d, num_segments=n_lig)

    za_c, zb_c = _sc_gather2(A_c, B_lc, cs2, cd2)
    rel_c = _sc_pos_rel(poc_x_c[:n_lig], x_t, cs2, cd2)
    ct_c = _edge_coef(za_c, zb_c, rel_c, _pad_attr(ep_cross, cross_edge_attr),
                      wd_c, Wc_c, w_x_c[:, 0])
    v_c = jax.ops.segment_sum(ct_c[:e_cross, :3], cd, num_segments=n_lig)

    v = v_l + v_c
    return jnp.mean((v - target) ** 2)


# speedup vs baseline: 1.4552x; 1.0018x over previous
import functools

import jax
import jax.numpy as jnp
from jax import lax
from jax.experimental import pallas as pl
from jax.experimental.pallas import tpu as pltpu
from jax.experimental.pallas import tpu_sc as plsc

_F32 = jnp.float32
_NW = 32
_LI = 128
_NSEG = 10240
_TRASH = 10000


def _sc_mesh():
    return plsc.VectorSubcoreMesh(core_axis_name="c", subcore_axis_name="s")


def _sc_gather2(ta, tb, idxa2, idxb2):
    nrows, li = idxa2.shape
    w = ta.shape[1]
    nper = nrows // _NW
    out = jax.ShapeDtypeStruct((nrows * li, w), _F32)

    @functools.partial(
        pl.kernel, mesh=_sc_mesh(),
        out_type=[out, out],
        scratch_types=[pltpu.VMEM((nper, li), jnp.int32),
                       pltpu.VMEM((nper, li), jnp.int32),
                       pltpu.VMEM((2, li, w), _F32),
                       pltpu.VMEM((2, li, w), _F32)]
                      + [pltpu.SemaphoreType.DMA] * 8,
    )
    def gk(ta_h, tb_h, ia_h, ib_h, oa_h, ob_h, ia_v, ib_v, bufa, bufb,
           ga0, ga1, gb0, gb1, wa0, wa1, wb0, wb1):
        gsa, gsb, wsa, wsb = (ga0, ga1), (gb0, gb1), (wa0, wa1), (wb0, wb1)
        wid = lax.axis_index("s") * 2 + lax.axis_index("c")
        rlo = wid * nper
        pltpu.sync_copy(ia_h.at[pl.ds(rlo, nper)], ia_v)
        pltpu.sync_copy(ib_h.at[pl.ds(rlo, nper)], ib_v)
        pltpu.async_copy(ta_h.at[ia_v.at[0]], bufa.at[0], gsa[0])
        pltpu.async_copy(tb_h.at[ib_v.at[0]], bufb.at[0], gsb[0])

        def pair(j0, carry):
            for b in (0, 1):
                j = j0 * 2 + b
                ob = 1 - b
                pltpu.make_async_copy(ta_h.at[ia_v.at[j]], bufa.at[b],
                                      gsa[b]).wait()
                pltpu.make_async_copy(tb_h.at[ib_v.at[j]], bufb.at[b],
                                      gsb[b]).wait()

                @pl.when(j >= 1)
                def _():
                    rw = (rlo + j - 1) * li
                    pltpu.make_async_copy(
                        bufa.at[ob], oa_h.at[pl.ds(rw, li)], wsa[ob]).wait()
                    pltpu.make_async_copy(
                        bufb.at[ob], ob_h.at[pl.ds(rw, li)], wsb[ob]).wait()

                @pl.when(j + 1 < nper)
                def _():
                    pltpu.async_copy(ta_h.at[ia_v.at[j + 1]], bufa.at[ob],
                                     gsa[ob])
                    pltpu.async_copy(tb_h.at[ib_v.at[j + 1]], bufb.at[ob],
                                     gsb[ob])

                rw = (rlo + j) * li
                pltpu.async_copy(bufa.at[b], oa_h.at[pl.ds(rw, li)], wsa[b])
                pltpu.async_copy(bufb.at[b], ob_h.at[pl.ds(rw, li)], wsb[b])
            return carry

        lax.fori_loop(0, nper // 2, pair, 0)
        rw = (rlo + nper - 1) * li
        lb = (nper - 1) % 2
        pltpu.make_async_copy(bufa.at[lb], oa_h.at[pl.ds(rw, li)],
                              wsa[lb]).wait()
        pltpu.make_async_copy(bufb.at[lb], ob_h.at[pl.ds(rw, li)],
                              wsb[lb]).wait()

    return gk(ta, tb, idxa2, idxb2)


def _sc_scatter_add(m, idx2, w):
    nrows, li = idx2.shape
    nper = nrows // _NW
    rpt = _NSEG // 16

    @functools.partial(
        pl.kernel, mesh=_sc_mesh(),
        out_type=jax.ShapeDtypeStruct((2, _NSEG, w), _F32),
        scratch_types=[pltpu.VMEM((nper, li), jnp.int32),
                       pltpu.VMEM((2, li, w), _F32),
                       pltpu.VMEM((32, w), _F32),
                       pltpu.VMEM_SHARED((_NSEG, w), _F32)]
                      + [pltpu.SemaphoreType.DMA] * 2,
    )
    def sk(m_hbm, idx_hbm, out_hbm, idx_v, mbuf, zbuf, shared, ls0, ls1):
        lsem = (ls0, ls1)
        c = lax.axis_index("c")
        sid = lax.axis_index("s")
        wid = sid * 2 + c

        def zb(i, carry):
            r = i // (w // 16)
            col = (i % (w // 16)) * 16
            zbuf[r, pl.ds(col, 16)] = jnp.zeros((16,), _F32)
            return carry

        lax.fori_loop(0, 32 * (w // 16), zb, 0)

        zlo = sid * rpt

        def zs(k, carry):
            pltpu.sync_copy(zbuf, shared.at[pl.ds(zlo + k * 32, 32)])
            return carry

        lax.fori_loop(0, rpt // 32, zs, 0)

        rlo = wid * nper
        pltpu.sync_copy(idx_hbm.at[pl.ds(rlo, nper)], idx_v)
        plsc.subcore_barrier()
        pltpu.async_copy(m_hbm.at[pl.ds(rlo * li, li)], mbuf.at[0], lsem[0])

        def pair(j0, carry):
            for b in (0, 1):
                j = j0 * 2 + b
                pltpu.make_async_copy(m_hbm.at[pl.ds((rlo + j) * li, li)],
                                      mbuf.at[b], lsem[b]).wait()

                @pl.when(j + 1 < nper)
                def _():
                    pltpu.async_copy(
                        m_hbm.at[pl.ds((rlo + j + 1) * li, li)],
                        mbuf.at[1 - b], lsem[1 - b])

                pltpu.sync_copy(mbuf.at[b], shared.at[idx_v.at[j]], add=True)
            return carry

        lax.fori_loop(0, nper // 2, pair, 0)
        plsc.subcore_barrier()

        def wr(k, carry):
            lo2 = zlo + k * li
            pltpu.sync_copy(shared.at[pl.ds(lo2, li)], mbuf.at[0])
            pltpu.sync_copy(mbuf.at[0], out_hbm.at[c, pl.ds(lo2, li)])
            return carry

        lax.fori_loop(0, rpt // li, wr, 0)

    return sk(m, idx2)


def _sc_pos_rel(pos_s, pos_d, idxs2, idxd2):
    shared = pos_s is pos_d
    ns = pos_s.shape[0]
    nd = pos_d.shape[0]
    nrows, li = idxs2.shape
    nper = nrows // _NW

    plane_scratch = [pltpu.VMEM((ns,), _F32)] * 3
    if not shared:
        plane_scratch += [pltpu.VMEM((nd,), _F32)] * 3
    ins = tuple(pos_s[:, i] for i in range(3))
    if not shared:
        ins += tuple(pos_d[:, i] for i in range(3))

    @functools.partial(
        pl.kernel, mesh=_sc_mesh(),
        compiler_params=pltpu.CompilerParams(needs_layout_passes=False),
        out_type=jax.ShapeDtypeStruct((nrows * li, 16), _F32),
        scratch_types=plane_scratch + [
            pltpu.VMEM((nper, li), jnp.int32),
            pltpu.VMEM((nper, li), jnp.int32),
            pltpu.VMEM((2, li, 16), _F32),
            pltpu.SemaphoreType.DMA, pltpu.SemaphoreType.DMA],
    )
    def pk(*refs):
        nplanes = 3 if shared else 6
        plane_h = refs[:nplanes]
        is_h, id_h, out_hbm = refs[nplanes], refs[nplanes + 1], refs[nplanes + 2]
        plane_v = refs[nplanes + 3:2 * nplanes + 3]
        is_v, id_v, rbuf = refs[-5], refs[-4], refs[-3]
        wsem = (refs[-2], refs[-1])
        if shared:
            src_v = dst_v = plane_v
        else:
            src_v, dst_v = plane_v[:3], plane_v[3:]

        wid = lax.axis_index("s") * 2 + lax.axis_index("c")
        rlo = wid * nper
        for h, v in zip(plane_h, plane_v):
            pltpu.sync_copy(h, v)
        pltpu.sync_copy(is_h.at[pl.ds(rlo, nper)], is_v)
        pltpu.sync_copy(id_h.at[pl.ds(rlo, nper)], id_v)

        def zr(i, carry):
            rbuf[i // li, i % li, pl.ds(0, 16)] = jnp.zeros((16,), _F32)
            return carry

        lax.fori_loop(0, 2 * li, zr, 0)
        iota = lax.iota(jnp.int32, 16)

        def pair(j0, carry):
            for b in (0, 1):
                j = j0 * 2 + b

                @pl.when(j >= 2)
                def _():
                    pltpu.make_async_copy(
                        rbuf.at[b],
                        out_hbm.at[pl.ds((rlo + j - 2) * li, li)],
                        wsem[b]).wait()

                for g in range(li // 16):
                    si = is_v[j, pl.ds(g * 16, 16)]
                    di = id_v[j, pl.ds(g * 16, 16)]
                    ridx = iota + (g * 16)
                    for comp in range(3):
                        vs = plsc.load_gather(src_v[comp], [si])
                        vd = plsc.load_gather(dst_v[comp], [di])
                        plsc.store_scatter(
                            rbuf.at[b],
                            [ridx, jnp.full((16,), comp, jnp.int32)],
                            vd - vs)
                pltpu.async_copy(rbuf.at[b],
                                 out_hbm.at[pl.ds((rlo + j) * li, li)],
                                 wsem[b])
            return carry

        lax.fori_loop(0, nper // 2, pair, 0)
        for b in (0, 1):
            j = nper - 2 + b
            pltpu.make_async_copy(
                rbuf.at[b], out_hbm.at[pl.ds((rlo + j) * li, li)],
                wsem[b]).wait()

    return pk(*ins, idxs2, idxd2)


def _node3_body(h_ref, add_ref, Win_ref, W1_ref, W2_ref, W3_ref,
                o1_ref, o2_ref, o3_ref, oh_ref):
    h = jnp.tanh(jnp.dot(h_ref[...], Win_ref[...],
                         preferred_element_type=_F32)) + add_ref[...]
    o1_ref[...] = jnp.dot(h, W1_ref[...], preferred_element_type=_F32)
    o2_ref[...] = jnp.dot(h, W2_ref[...], preferred_element_type=_F32)
    o3_ref[...] = jnp.dot(h, W3_ref[...], preferred_element_type=_F32)
    oh_ref[...] = h


def _node3(h, add, Win, W1, W2, W3, bn=2000):
    n, dh = h.shape
    row = pl.BlockSpec((bn, dh), lambda i: (i, 0))
    wsp = pl.BlockSpec((dh, dh), lambda i: (0, 0))
    out = jax.ShapeDtypeStruct((n, dh), _F32)
    return pl.pallas_call(
        _node3_body,
        grid=(n // bn,),
        in_specs=[row, row, wsp, wsp, wsp, wsp],
        out_specs=[row, row, row, row],
        out_shape=[out, out, out, out],
    )(h, add, Win, W1, W2, W3)


def _edge_common(za, zb, rel16, attr, wd, Wc):
    d2 = jnp.sum(rel16 * rel16, axis=1)
    z = (za + zb
         + d2[:, None] * wd[None, :]
         + jnp.dot(attr, Wc, preferred_element_type=_F32))
    return jax.nn.relu(z)


def _edge_coef_body(za_ref, zb_ref, rel_ref, attr_ref, wd_ref, Wc_ref,
                    wx_ref, out_ref):
    m = _edge_common(za_ref[...], zb_ref[...], rel_ref[...], attr_ref[...],
                     wd_ref[...], Wc_ref[...])
    coef = jnp.tanh(jnp.sum(m * wx_ref[...][None, :], axis=1))
    out_ref[...] = rel_ref[...] * coef[:, None]


def _edge_coef(za, zb, rel16, attr, wd, Wc, wx, be=4096):
    e = za.shape[0]
    de = attr.shape[1]
    return pl.pallas_call(
        _edge_coef_body,
        grid=(e // be,),
        in_specs=[pl.BlockSpec((be, 128), lambda i: (i, 0)),
                  pl.BlockSpec((be, 128), lambda i: (i, 0)),
                  pl.BlockSpec((be, 16), lambda i: (i, 0)),
                  pl.BlockSpec((be, de), lambda i: (i, 0)),
                  pl.BlockSpec((128,), lambda i: (0,)),
                  pl.BlockSpec((de, 128), lambda i: (0, 0)),
                  pl.BlockSpec((128,), lambda i: (0,))],
        out_specs=pl.BlockSpec((be, 16), lambda i: (i, 0)),
        out_shape=jax.ShapeDtypeStruct((e, 16), _F32),
    )(za, zb, rel16, attr, wd, Wc, wx)


def _edge_msg_body(za_ref, zb_ref, rel_ref, attr_ref, wd_ref, Wc_ref, m_ref):
    m_ref[...] = _edge_common(za_ref[...], zb_ref[...], rel_ref[...],
                              attr_ref[...], wd_ref[...], Wc_ref[...])


def _edge_msg(za, zb, rel16, attr, wd, Wc, be=4096):
    e = za.shape[0]
    de = attr.shape[1]
    return pl.pallas_call(
        _edge_msg_body,
        grid=(e // be,),
        in_specs=[pl.BlockSpec((be, 128), lambda i: (i, 0)),
                  pl.BlockSpec((be, 128), lambda i: (i, 0)),
                  pl.BlockSpec((be, 16), lambda i: (i, 0)),
                  pl.BlockSpec((be, de), lambda i: (i, 0)),
                  pl.BlockSpec((128,), lambda i: (0,)),
                  pl.BlockSpec((de, 128), lambda i: (0, 0))],
        out_specs=pl.BlockSpec((be, 128), lambda i: (i, 0)),
        out_shape=jax.ShapeDtypeStruct((e, 128), _F32),
    )(za, zb, rel16, attr, wd, Wc)


def _upd_proj_body(h_ref, s_ref, W1_ref, W2_ref, o_ref):
    hp = h_ref[...] + jnp.dot(s_ref[...], W1_ref[...],
                              preferred_element_type=_F32)
    o_ref[...] = jnp.dot(hp, W2_ref[...], preferred_element_type=_F32)


def _upd_proj(h, sm, W1, W2, bn=2000):
    n, dh = h.shape
    row = pl.BlockSpec((bn, dh), lambda i: (i, 0))
    wsp = pl.BlockSpec((dh, dh), lambda i: (0, 0))
    return pl.pallas_call(
        _upd_proj_body,
        grid=(n // bn,),
        in_specs=[row, row, wsp, wsp],
        out_specs=row,
        out_shape=jax.ShapeDtypeStruct((n, dh), _F32),
    )(h, sm, W1, W2)


def _pad_rows(e_pad, idx, fill):
    npad = e_pad - idx.shape[0]
    if isinstance(fill, int) and fill == _TRASH:
        pad = _TRASH + (jnp.arange(npad, dtype=jnp.int32) % (_NSEG - _TRASH))
    else:
        pad = jnp.full((npad,), fill, jnp.int32)
    return jnp.concatenate([idx, pad]).reshape(e_pad // _LI, _LI)


def _pad_attr(e_pad, attr):
    return jnp.concatenate(
        [attr, jnp.zeros((e_pad - attr.shape[0], attr.shape[1]), _F32)])


def kernel(lig_x, lig_h, poc_x, poc_h, lig_edge_index, lig_edge_attr,
           poc_edge_index, poc_edge_attr, cross_edge_index, cross_edge_attr,
           lig_batch, poc_batch, W_in, w_t, W_in_p, W_m1, w_x_l, W_p1, W_p2,
           W_c1, w_x_c):
    n_lig = lig_x.shape[0]
    n_poc = poc_x.shape[0]
    dh = lig_h.shape[1]
    n_graphs = 200
    e_lig = lig_edge_index.shape[1]
    e_poc = poc_edge_index.shape[1]
    e_cross = cross_edge_index.shape[1]
    blk = _NW * 8 * _LI
    ep_lig = ((e_lig + blk - 1) // blk) * blk
    ep_poc = ((e_poc + blk - 1) // blk) * blk
    ep_cross = ((e_cross + blk - 1) // blk) * blk

    k1, k2 = jax.random.split(jax.random.key(42))
    t_per_graph = jax.random.uniform(k1, (n_graphs,), dtype=_F32)
    t_atom = t_per_graph[lig_batch]
    x0 = jax.random.normal(k2, lig_x.shape, dtype=_F32)

    poc_sum = jax.ops.segment_sum(poc_x, poc_batch, num_segments=n_graphs)
    poc_count = jnp.maximum(
        jax.ops.segment_sum(jnp.ones((n_poc, 1), dtype=_F32), poc_batch,
                            num_segments=n_graphs), 1.0)
    poc_center = poc_sum / poc_count
    poc_x_c = poc_x - poc_center[poc_batch]
    lig_x1_c = lig_x - poc_center[lig_batch]
    t_col = t_atom[:, None]
    x_t = (1.0 - t_col) * x0 + t_col * lig_x1_c
    target = lig_x1_c - x0

    Wa_m, Wb_m, wd_m, Wc_m = W_m1[:dh], W_m1[dh:2*dh], W_m1[2*dh], W_m1[2*dh+1:]
    Wa_p, Wb_p, wd_p, Wc_p = W_p1[:dh], W_p1[dh:2*dh], W_p1[2*dh], W_p1[2*dh+1:]
    Wa_c, Wb_c, wd_c, Wc_c = W_c1[:dh], W_c1[dh:2*dh], W_c1[2*dh], W_c1[2*dh+1:]

    T = t_col * w_t[None, :]
    A_l, B_l, B_lc, _ = _node3(lig_h, T, W_in, Wa_m, Wb_m, Wb_c)
    Zp = jnp.zeros((n_poc, dh), dtype=_F32)
    A_p, B_p, _, h_p = _node3(poc_h, Zp, W_in_p, Wa_p, Wb_p, Wa_c)

    ps, pd = poc_edge_index[0], poc_edge_index[1]
    s, d = lig_edge_index[0], lig_edge_index[1]
    cs, cd = cross_edge_index[0], cross_edge_index[1]

    s2 = _pad_rows(ep_lig, s, 0)
    d2i = _pad_rows(ep_lig, d, 0)
    dsc = _pad_rows(ep_lig, d, _TRASH)
    ps2 = _pad_rows(ep_poc, ps, 0)
    pd2 = _pad_rows(ep_poc, pd, 0)
    pd_spread = jnp.where(pd < n_lig, pd, n_lig + pd % (_NSEG - n_lig))
    pdsc = _pad_rows(ep_poc, pd_spread, _TRASH)
    cs2 = _pad_rows(ep_cross, cs, 0)
    cd2 = _pad_rows(ep_cross, cd, 0)
    cdsc = _pad_rows(ep_cross, cd, _TRASH)

    za_p, zb_p = _sc_gather2(A_p, B_p, ps2, pd2)
    rel_p = _sc_pos_rel(poc_x_c, poc_x_c, ps2, pd2)
    m_p = _edge_msg(za_p, zb_p, rel_p, _pad_attr(ep_poc, poc_edge_attr),
                    wd_p, Wc_p)
    S2 = _sc_scatter_add(m_p, pdsc, 128)
    seg = S2[0, :n_lig] + S2[1, :n_lig]
    A_c = _upd_proj(h_p[:n_lig], seg, W_p2, Wa_c)

    za_l, zb_l = _sc_gather2(A_l, B_l, s2, d2i)
    rel_l = _sc_pos_rel(x_t, x_t, s2, d2i)
    ct_l = _edge_coef(za_l, zb_l, rel_l, _pad_attr(ep_lig, lig_edge_attr),
                      wd_m, Wc_m, w_x_l[:, 0])
    v_l = jax.ops.segment_sum(ct_l[:e_lig, :3], d, num_segments=n_lig)

    za_c, zb_c = _sc_gather2(A_c, B_lc, cs2, cd2)
    rel_c = _sc_pos_rel(poc_x_c[:n_lig], x_t, cs2, cd2)
    ct_c = _edge_coef(za_c, zb_c, rel_c, _pad_attr(ep_cross, cross_edge_attr),
                      wd_c, Wc_c, w_x_c[:, 0])
    v_c = jax.ops.segment_sum(ct_c[:e_cross, :3], cd, num_segments=n_lig)

    v = v_l + v_c
    return jnp.mean((v - target) ** 2)


# speedup vs baseline: 1.4553x; 1.0001x over previous
import functools

import jax
import jax.numpy as jnp
from jax import lax
from jax.experimental import pallas as pl
from jax.experimental.pallas import tpu as pltpu
from jax.experimental.pallas import tpu_sc as plsc

_F32 = jnp.float32
_NW = 32
_LI = 128
_NSEG = 10240
_TRASH = 10000


def _sc_mesh():
    return plsc.VectorSubcoreMesh(core_axis_name="c", subcore_axis_name="s")


def _sc_gather2(ta, tb, idxa2, idxb2):
    nrows, li = idxa2.shape
    w = ta.shape[1]
    nper = nrows // _NW
    out = jax.ShapeDtypeStruct((nrows * li, w), _F32)

    @functools.partial(
        pl.kernel, mesh=_sc_mesh(),
        out_type=[out, out],
        scratch_types=[pltpu.VMEM((nper, li), jnp.int32),
                       pltpu.VMEM((nper, li), jnp.int32),
                       pltpu.VMEM((2, li, w), _F32),
                       pltpu.VMEM((2, li, w), _F32)]
                      + [pltpu.SemaphoreType.DMA] * 8,
    )
    def gk(ta_h, tb_h, ia_h, ib_h, oa_h, ob_h, ia_v, ib_v, bufa, bufb,
           ga0, ga1, gb0, gb1, wa0, wa1, wb0, wb1):
        gsa, gsb, wsa, wsb = (ga0, ga1), (gb0, gb1), (wa0, wa1), (wb0, wb1)
        wid = lax.axis_index("s") * 2 + lax.axis_index("c")
        rlo = wid * nper
        pltpu.sync_copy(ia_h.at[pl.ds(rlo, nper)], ia_v)
        pltpu.sync_copy(ib_h.at[pl.ds(rlo, nper)], ib_v)
        pltpu.async_copy(ta_h.at[ia_v.at[0]], bufa.at[0], gsa[0])
        pltpu.async_copy(tb_h.at[ib_v.at[0]], bufb.at[0], gsb[0])

        def pair(j0, carry):
            for b in (0, 1):
                j = j0 * 2 + b
                ob = 1 - b
                pltpu.make_async_copy(ta_h.at[ia_v.at[j]], bufa.at[b],
                                      gsa[b]).wait()
                pltpu.make_async_copy(tb_h.at[ib_v.at[j]], bufb.at[b],
                                      gsb[b]).wait()

                @pl.when(j >= 1)
                def _():
                    rw = (rlo + j - 1) * li
                    pltpu.make_async_copy(
                        bufa.at[ob], oa_h.at[pl.ds(rw, li)], wsa[ob]).wait()
                    pltpu.make_async_copy(
                        bufb.at[ob], ob_h.at[pl.ds(rw, li)], wsb[ob]).wait()

                @pl.when(j + 1 < nper)
                def _():
                    pltpu.async_copy(ta_h.at[ia_v.at[j + 1]], bufa.at[ob],
                                     gsa[ob])
                    pltpu.async_copy(tb_h.at[ib_v.at[j + 1]], bufb.at[ob],
                                     gsb[ob])

                rw = (rlo + j) * li
                pltpu.async_copy(bufa.at[b], oa_h.at[pl.ds(rw, li)], wsa[b])
                pltpu.async_copy(bufb.at[b], ob_h.at[pl.ds(rw, li)], wsb[b])
            return carry

        lax.fori_loop(0, nper // 2, pair, 0)
        rw = (rlo + nper - 1) * li
        lb = (nper - 1) % 2
        pltpu.make_async_copy(bufa.at[lb], oa_h.at[pl.ds(rw, li)],
                              wsa[lb]).wait()
        pltpu.make_async_copy(bufb.at[lb], ob_h.at[pl.ds(rw, li)],
                              wsb[lb]).wait()

    return gk(ta, tb, idxa2, idxb2)


def _sc_scatter_add(m, idx2, w):
    nrows, li = idx2.shape
    nper = nrows // _NW
    rpt = _NSEG // 16

    @functools.partial(
        pl.kernel, mesh=_sc_mesh(),
        out_type=jax.ShapeDtypeStruct((2, _NSEG, w), _F32),
        scratch_types=[pltpu.VMEM((nper, li), jnp.int32),
                       pltpu.VMEM((2, li, w), _F32),
                       pltpu.VMEM((32, w), _F32),
                       pltpu.VMEM_SHARED((_NSEG, w), _F32)]
                      + [pltpu.SemaphoreType.DMA] * 4,
    )
    def sk(m_hbm, idx_hbm, out_hbm, idx_v, mbuf, zbuf, shared, ls0, ls1,
           ss0, ss1):
        lsem = (ls0, ls1)
        ssem = (ss0, ss1)
        c = lax.axis_index("c")
        sid = lax.axis_index("s")
        wid = sid * 2 + c

        def zb(i, carry):
            r = i // (w // 16)
            col = (i % (w // 16)) * 16
            zbuf[r, pl.ds(col, 16)] = jnp.zeros((16,), _F32)
            return carry

        lax.fori_loop(0, 32 * (w // 16), zb, 0)

        zlo = sid * rpt

        def zs(k, carry):
            pltpu.sync_copy(zbuf, shared.at[pl.ds(zlo + k * 32, 32)])
            return carry

        lax.fori_loop(0, rpt // 32, zs, 0)

        rlo = wid * nper
        pltpu.sync_copy(idx_hbm.at[pl.ds(rlo, nper)], idx_v)
        plsc.subcore_barrier()
        pltpu.async_copy(m_hbm.at[pl.ds(rlo * li, li)], mbuf.at[0], lsem[0])

        def pair(j0, carry):
            for b in (0, 1):
                j = j0 * 2 + b
                ob = 1 - b
                pltpu.make_async_copy(m_hbm.at[pl.ds((rlo + j) * li, li)],
                                      mbuf.at[b], lsem[b]).wait()

                @pl.when(j >= 1)
                def _():
                    pltpu.make_async_copy(
                        mbuf.at[ob], shared.at[idx_v.at[j - 1]],
                        ssem[ob]).wait()

                @pl.when(j + 1 < nper)
                def _():
                    pltpu.async_copy(
                        m_hbm.at[pl.ds((rlo + j + 1) * li, li)],
                        mbuf.at[ob], lsem[ob])

                pltpu.async_copy(mbuf.at[b], shared.at[idx_v.at[j]],
                                 ssem[b], add=True)
            return carry

        lax.fori_loop(0, nper // 2, pair, 0)
        pltpu.make_async_copy(mbuf.at[(nper - 1) % 2],
                              shared.at[idx_v.at[nper - 1]],
                              ssem[(nper - 1) % 2]).wait()
        plsc.subcore_barrier()

        def wr(k, carry):
            lo2 = zlo + k * li
            pltpu.sync_copy(shared.at[pl.ds(lo2, li)], mbuf.at[0])
            pltpu.sync_copy(mbuf.at[0], out_hbm.at[c, pl.ds(lo2, li)])
            return carry

        lax.fori_loop(0, rpt // li, wr, 0)

    return sk(m, idx2)


def _sc_pos_rel(pos_s, pos_d, idxs2, idxd2):
    shared = pos_s is pos_d
    ns = pos_s.shape[0]
    nd = pos_d.shape[0]
    nrows, li = idxs2.shape
    nper = nrows // _NW

    plane_scratch = [pltpu.VMEM((ns,), _F32)] * 3
    if not shared:
        plane_scratch += [pltpu.VMEM((nd,), _F32)] * 3
    ins = tuple(pos_s[:, i] for i in range(3))
    if not shared:
        ins += tuple(pos_d[:, i] for i in range(3))

    @functools.partial(
        pl.kernel, mesh=_sc_mesh(),
        compiler_params=pltpu.CompilerParams(needs_layout_passes=False),
        out_type=jax.ShapeDtypeStruct((nrows * li, 16), _F32),
        scratch_types=plane_scratch + [
            pltpu.VMEM((nper, li), jnp.int32),
            pltpu.VMEM((nper, li), jnp.int32),
            pltpu.VMEM((2, li, 16), _F32),
            pltpu.SemaphoreType.DMA, pltpu.SemaphoreType.DMA],
    )
    def pk(*refs):
        nplanes = 3 if shared else 6
        plane_h = refs[:nplanes]
        is_h, id_h, out_hbm = refs[nplanes], refs[nplanes + 1], refs[nplanes + 2]
        plane_v = refs[nplanes + 3:2 * nplanes + 3]
        is_v, id_v, rbuf = refs[-5], refs[-4], refs[-3]
        wsem = (refs[-2], refs[-1])
        if shared:
            src_v = dst_v = plane_v
        else:
            src_v, dst_v = plane_v[:3], plane_v[3:]

        wid = lax.axis_index("s") * 2 + lax.axis_index("c")
        rlo = wid * nper
        for h, v in zip(plane_h, plane_v):
            pltpu.sync_copy(h, v)
        pltpu.sync_copy(is_h.at[pl.ds(rlo, nper)], is_v)
        pltpu.sync_copy(id_h.at[pl.ds(rlo, nper)], id_v)

        def zr(i, carry):
            rbuf[i // li, i % li, pl.ds(0, 16)] = jnp.zeros((16,), _F32)
            return carry

        lax.fori_loop(0, 2 * li, zr, 0)
        iota = lax.iota(jnp.int32, 16)

        def pair(j0, carry):
            for b in (0, 1):
                j = j0 * 2 + b

                @pl.when(j >= 2)
                def _():
                    pltpu.make_async_copy(
                        rbuf.at[b],
                        out_hbm.at[pl.ds((rlo + j - 2) * li, li)],
                        wsem[b]).wait()

                for g in range(li // 16):
                    si = is_v[j, pl.ds(g * 16, 16)]
                    di = id_v[j, pl.ds(g * 16, 16)]
                    ridx = iota + (g * 16)
                    for comp in range(3):
                        vs = plsc.load_gather(src_v[comp], [si])
                        vd = plsc.load_gather(dst_v[comp], [di])
                        plsc.store_scatter(
                            rbuf.at[b],
                            [ridx, jnp.full((16,), comp, jnp.int32)],
                            vd - vs)
                pltpu.async_copy(rbuf.at[b],
                                 out_hbm.at[pl.ds((rlo + j) * li, li)],
                                 wsem[b])
            return carry

        lax.fori_loop(0, nper // 2, pair, 0)
        for b in (0, 1):
            j = nper - 2 + b
            pltpu.make_async_copy(
                rbuf.at[b], out_hbm.at[pl.ds((rlo + j) * li, li)],
                wsem[b]).wait()

    return pk(*ins, idxs2, idxd2)


def _node3_body(h_ref, add_ref, Win_ref, W1_ref, W2_ref, W3_ref,
                o1_ref, o2_ref, o3_ref, oh_ref):
    h = jnp.tanh(jnp.dot(h_ref[...], Win_ref[...],
                         preferred_element_type=_F32)) + add_ref[...]
    o1_ref[...] = jnp.dot(h, W1_ref[...], preferred_element_type=_F32)
    o2_ref[...] = jnp.dot(h, W2_ref[...], preferred_element_type=_F32)
    o3_ref[...] = jnp.dot(h, W3_ref[...], preferred_element_type=_F32)
    oh_ref[...] = h


def _node3(h, add, Win, W1, W2, W3, bn=2000):
    n, dh = h.shape
    row = pl.BlockSpec((bn, dh), lambda i: (i, 0))
    wsp = pl.BlockSpec((dh, dh), lambda i: (0, 0))
    out = jax.ShapeDtypeStruct((n, dh), _F32)
    return pl.pallas_call(
        _node3_body,
        grid=(n // bn,),
        in_specs=[row, row, wsp, wsp, wsp, wsp],
        out_specs=[row, row, row, row],
        out_shape=[out, out, out, out],
    )(h, add, Win, W1, W2, W3)


def _edge_common(za, zb, rel16, attr, wd, Wc):
    d2 = jnp.sum(rel16 * rel16, axis=1)
    z = (za + zb
         + d2[:, None] * wd[None, :]
         + jnp.dot(attr, Wc, preferred_element_type=_F32))
    return jax.nn.relu(z)


def _edge_coef_body(za_ref, zb_ref, rel_ref, attr_ref, wd_ref, Wc_ref,
                    wx_ref, out_ref):
    m = _edge_common(za_ref[...], zb_ref[...], rel_ref[...], attr_ref[...],
                     wd_ref[...], Wc_ref[...])
    coef = jnp.tanh(jnp.sum(m * wx_ref[...][None, :], axis=1))
    out_ref[...] = rel_ref[...] * coef[:, None]


def _edge_coef(za, zb, rel16, attr, wd, Wc, wx, be=4096):
    e = za.shape[0]
    de = attr.shape[1]
    return pl.pallas_call(
        _edge_coef_body,
        grid=(e // be,),
        in_specs=[pl.BlockSpec((be, 128), lambda i: (i, 0)),
                  pl.BlockSpec((be, 128), lambda i: (i, 0)),
                  pl.BlockSpec((be, 16), lambda i: (i, 0)),
                  pl.BlockSpec((be, de), lambda i: (i, 0)),
                  pl.BlockSpec((128,), lambda i: (0,)),
                  pl.BlockSpec((de, 128), lambda i: (0, 0)),
                  pl.BlockSpec((128,), lambda i: (0,))],
        out_specs=pl.BlockSpec((be, 16), lambda i: (i, 0)),
        out_shape=jax.ShapeDtypeStruct((e, 16), _F32),
    )(za, zb, rel16, attr, wd, Wc, wx)


def _edge_msg_body(za_ref, zb_ref, rel_ref, attr_ref, wd_ref, Wc_ref, m_ref):
    m_ref[...] = _edge_common(za_ref[...], zb_ref[...], rel_ref[...],
                              attr_ref[...], wd_ref[...], Wc_ref[...])


def _edge_msg(za, zb, rel16, attr, wd, Wc, be=4096):
    e = za.shape[0]
    de = attr.shape[1]
    return pl.pallas_call(
        _edge_msg_body,
        grid=(e // be,),
        in_specs=[pl.BlockSpec((be, 128), lambda i: (i, 0)),
                  pl.BlockSpec((be, 128), lambda i: (i, 0)),
                  pl.BlockSpec((be, 16), lambda i: (i, 0)),
                  pl.BlockSpec((be, de), lambda i: (i, 0)),
                  pl.BlockSpec((128,), lambda i: (0,)),
                  pl.BlockSpec((de, 128), lambda i: (0, 0))],
        out_specs=pl.BlockSpec((be, 128), lambda i: (i, 0)),
        out_shape=jax.ShapeDtypeStruct((e, 128), _F32),
    )(za, zb, rel16, attr, wd, Wc)


def _upd_proj_body(h_ref, s_ref, W1_ref, W2_ref, o_ref):
    hp = h_ref[...] + jnp.dot(s_ref[...], W1_ref[...],
                              preferred_element_type=_F32)
    o_ref[...] = jnp.dot(hp, W2_ref[...], preferred_element_type=_F32)


def _upd_proj(h, sm, W1, W2, bn=2000):
    n, dh = h.shape
    row = pl.BlockSpec((bn, dh), lambda i: (i, 0))
    wsp = pl.BlockSpec((dh, dh), lambda i: (0, 0))
    return pl.pallas_call(
        _upd_proj_body,
        grid=(n // bn,),
        in_specs=[row, row, wsp, wsp],
        out_specs=row,
        out_shape=jax.ShapeDtypeStruct((n, dh), _F32),
    )(h, sm, W1, W2)


def _pad_rows(e_pad, idx, fill):
    npad = e_pad - idx.shape[0]
    if isinstance(fill, int) and fill == _TRASH:
        pad = _TRASH + (jnp.arange(npad, dtype=jnp.int32) % (_NSEG - _TRASH))
    else:
        pad = jnp.full((npad,), fill, jnp.int32)
    return jnp.concatenate([idx, pad]).reshape(e_pad // _LI, _LI)


def _pad_attr(e_pad, attr):
    return jnp.concatenate(
        [attr, jnp.zeros((e_pad - attr.shape[0], attr.shape[1]), _F32)])


def kernel(lig_x, lig_h, poc_x, poc_h, lig_edge_index, lig_edge_attr,
           poc_edge_index, poc_edge_attr, cross_edge_index, cross_edge_attr,
           lig_batch, poc_batch, W_in, w_t, W_in_p, W_m1, w_x_l, W_p1, W_p2,
           W_c1, w_x_c):
    n_lig = lig_x.shape[0]
    n_poc = poc_x.shape[0]
    dh = lig_h.shape[1]
    n_graphs = 200
    e_lig = lig_edge_index.shape[1]
    e_poc = poc_edge_index.shape[1]
    e_cross = cross_edge_index.shape[1]
    blk = _NW * 8 * _LI
    ep_lig = ((e_lig + blk - 1) // blk) * blk
    ep_poc = ((e_poc + blk - 1) // blk) * blk
    ep_cross = ((e_cross + blk - 1) // blk) * blk

    k1, k2 = jax.random.split(jax.random.key(42))
    t_per_graph = jax.random.uniform(k1, (n_graphs,), dtype=_F32)
    t_atom = t_per_graph[lig_batch]
    x0 = jax.random.normal(k2, lig_x.shape, dtype=_F32)

    poc_sum = jax.ops.segment_sum(poc_x, poc_batch, num_segments=n_graphs)
    poc_count = jnp.maximum(
        jax.ops.segment_sum(jnp.ones((n_poc, 1), dtype=_F32), poc_batch,
                            num_segments=n_graphs), 1.0)
    poc_center = poc_sum / poc_count
    poc_x_c = poc_x - poc_center[poc_batch]
    lig_x1_c = lig_x - poc_center[lig_batch]
    t_col = t_atom[:, None]
    x_t = (1.0 - t_col) * x0 + t_col * lig_x1_c
    target = lig_x1_c - x0

    Wa_m, Wb_m, wd_m, Wc_m = W_m1[:dh], W_m1[dh:2*dh], W_m1[2*dh], W_m1[2*dh+1:]
    Wa_p, Wb_p, wd_p, Wc_p = W_p1[:dh], W_p1[dh:2*dh], W_p1[2*dh], W_p1[2*dh+1:]
    Wa_c, Wb_c, wd_c, Wc_c = W_c1[:dh], W_c1[dh:2*dh], W_c1[2*dh], W_c1[2*dh+1:]

    T = t_col * w_t[None, :]
    A_l, B_l, B_lc, _ = _node3(lig_h, T, W_in, Wa_m, Wb_m, Wb_c)
    Zp = jnp.zeros((n_poc, dh), dtype=_F32)
    A_p, B_p, _, h_p = _node3(poc_h, Zp, W_in_p, Wa_p, Wb_p, Wa_c)

    ps, pd = poc_edge_index[0], poc_edge_index[1]
    s, d = lig_edge_index[0], lig_edge_index[1]
    cs, cd = cross_edge_index[0], cross_edge_index[1]

    s2 = _pad_rows(ep_lig, s, 0)
    d2i = _pad_rows(ep_lig, d, 0)
    dsc = _pad_rows(ep_lig, d, _TRASH)
    ps2 = _pad_rows(ep_poc, ps, 0)
    pd2 = _pad_rows(ep_poc, pd, 0)
    pd_spread = jnp.where(pd < n_lig, pd, n_lig + pd % (_NSEG - n_lig))
    pdsc = _pad_rows(ep_poc, pd_spread, _TRASH)
    cs2 = _pad_rows(ep_cross, cs, 0)
    cd2 = _pad_rows(ep_cross, cd, 0)
    cdsc = _pad_rows(ep_cross, cd, _TRASH)

    za_p, zb_p = _sc_gather2(A_p, B_p, ps2, pd2)
    rel_p = _sc_pos_rel(poc_x_c, poc_x_c, ps2, pd2)
    m_p = _edge_msg(za_p, zb_p, rel_p, _pad_attr(ep_poc, poc_edge_attr),
                    wd_p, Wc_p)
    S2 = _sc_scatter_add(m_p, pdsc, 128)
    seg = S2[0, :n_lig] + S2[1, :n_lig]
    A_c = _upd_proj(h_p[:n_lig], seg, W_p2, Wa_c)

    za_l, zb_l = _sc_gather2(A_l, B_l, s2, d2i)
    rel_l = _sc_pos_rel(x_t, x_t, s2, d2i)
    ct_l = _edge_coef(za_l, zb_l, rel_l, _pad_attr(ep_lig, lig_edge_attr),
                      wd_m, Wc_m, w_x_l[:, 0])
    v_l = jax.ops.segment_sum(ct_l[:e_lig, :3], d, num_segments=n_lig)

    za_c, zb_c = _sc_gather2(A_c, B_lc, cs2, cd2)
    rel_c = _sc_pos_rel(poc_x_c[:n_lig], x_t, cs2, cd2)
    ct_c = _edge_coef(za_c, zb_c, rel_c, _pad_attr(ep_cross, cross_edge_attr),
                      wd_c, Wc_c, w_x_c[:, 0])
    v_c = jax.ops.segment_sum(ct_c[:e_cross, :3], cd, num_segments=n_lig)

    v = v_l + v_c
    return jnp.mean((v - target) ** 2)
